# Initial kernel scaffold; baseline (speedup 1.0000x reference)
import jax
import jax.numpy as jnp
from jax.experimental import pallas as pl


def kernel(arm_loc_data, arm_conf_data, loc_t, conf_t):
    raise NotImplementedError("write your pallas kernel here")


# speedup vs baseline: 7.3175x; 7.3175x over previous
import jax
import jax.numpy as jnp
from jax.experimental import pallas as pl


def _k1_body(conf_ref, ct_ref, loc_ref, loct_ref, raw_ref, lossl_ref):
    x = conf_ref[0]
    ct = ct_ref[0]
    xmax = jnp.max(x, axis=0, keepdims=True)
    e = jnp.exp(x - xmax)
    s = jnp.sum(e, axis=0, keepdims=True)
    cls = jax.lax.broadcasted_iota(jnp.int32, x.shape, 0)
    xt = jnp.sum(jnp.where(cls == ct, x, 0.0), axis=0, keepdims=True)
    raw_ref[0] = jnp.log(s) + xmax - xt

    d = loc_ref[0] - loct_ref[0]
    ad = jnp.abs(d)
    sl1 = jnp.where(ad < 1.0, 0.5 * d * d, ad - 0.5)
    posf = (ct > 0).astype(jnp.float32)
    part = jnp.sum(sl1 * posf)

    b = pl.program_id(0)

    @pl.when(b == 0)
    def _():
        lossl_ref[...] = jnp.zeros((1, 1), jnp.float32)

    lossl_ref[...] += part.reshape(1, 1)


def _k2_body(raw_ref, ct_ref, lossl_ref, outl_ref, outc_ref):
    raw = raw_ref[...]
    ct = ct_ref[...]
    nb, npr = raw.shape
    pos = ct > 0
    posf = pos.astype(jnp.float32)
    num_pos = jnp.sum(posf, axis=1, keepdims=True)
    k = jnp.minimum(3 * num_pos.astype(jnp.int32), npr - 1)

    lossc = jnp.where(pos, 0.0, raw)
    bits = jax.lax.bitcast_convert_type(lossc, jnp.int32)

    lo = jnp.zeros((nb, 1), jnp.int32)
    hi = jnp.full((nb, 1), 0x7F800000, jnp.int32)

    def vsearch(_, lohi):
        lo, hi = lohi
        mid = lo + ((hi - lo) >> 1)
        cnt = jnp.sum((bits > mid).astype(jnp.int32), axis=1, keepdims=True)
        below = cnt < k
        return jnp.where(below, lo, mid + 1), jnp.where(below, mid, hi)

    lo, hi = jax.lax.fori_loop(0, 31, vsearch, (lo, hi))
    t = hi
    gt = bits > t
    cnt_gt = jnp.sum(gt.astype(jnp.int32), axis=1, keepdims=True)
    need = k - cnt_gt
    eq = bits == t
    eqi = eq.astype(jnp.int32)
    idx = jax.lax.broadcasted_iota(jnp.int32, raw.shape, 1)

    lo2 = jnp.zeros((nb, 1), jnp.int32)
    hi2 = jnp.full((nb, 1), npr, jnp.int32)

    def isearch(_, lohi):
        lo2, hi2 = lohi
        mid = lo2 + ((hi2 - lo2) >> 1)
        c = jnp.sum(jnp.where(idx < mid, eqi, 0), axis=1, keepdims=True)
        ok = c >= need
        return jnp.where(ok, lo2, mid + 1), jnp.where(ok, mid, hi2)

    lo2, hi2 = jax.lax.fori_loop(0, 15, isearch, (lo2, hi2))
    neg = gt | (eq & (idx < hi2))
    maskf = jnp.where(pos | neg, 1.0, 0.0)
    loss_c_sum = jnp.sum(raw * maskf)
    n_total = jnp.sum(num_pos)
    outl_ref[...] = lossl_ref[...] / n_total
    outc_ref[...] = (loss_c_sum / n_total).reshape(1, 1)


def kernel(arm_loc_data, arm_conf_data, loc_t, conf_t):
    nb, npr, nc = arm_conf_data.shape
    conf_tr = jnp.transpose(arm_conf_data, (0, 2, 1))
    loc_tr = jnp.transpose(arm_loc_data, (0, 2, 1))
    loct_tr = jnp.transpose(loc_t, (0, 2, 1))
    ct3 = conf_t.reshape(nb, 1, npr)

    raw3, lossl = pl.pallas_call(
        _k1_body,
        grid=(nb,),
        in_specs=[
            pl.BlockSpec((1, nc, npr), lambda b: (b, 0, 0)),
            pl.BlockSpec((1, 1, npr), lambda b: (b, 0, 0)),
            pl.BlockSpec((1, 4, npr), lambda b: (b, 0, 0)),
            pl.BlockSpec((1, 4, npr), lambda b: (b, 0, 0)),
        ],
        out_specs=[
            pl.BlockSpec((1, 1, npr), lambda b: (b, 0, 0)),
            pl.BlockSpec((1, 1), lambda b: (0, 0)),
        ],
        out_shape=[
            jax.ShapeDtypeStruct((nb, 1, npr), jnp.float32),
            jax.ShapeDtypeStruct((1, 1), jnp.float32),
        ],
    )(conf_tr, ct3, loc_tr, loct_tr)

    outl, outc = pl.pallas_call(
        _k2_body,
        in_specs=[
            pl.BlockSpec((nb, npr), lambda: (0, 0)),
            pl.BlockSpec((nb, npr), lambda: (0, 0)),
            pl.BlockSpec((1, 1), lambda: (0, 0)),
        ],
        out_specs=[
            pl.BlockSpec((1, 1), lambda: (0, 0)),
            pl.BlockSpec((1, 1), lambda: (0, 0)),
        ],
        out_shape=[
            jax.ShapeDtypeStruct((1, 1), jnp.float32),
            jax.ShapeDtypeStruct((1, 1), jnp.float32),
        ],
    )(raw3.reshape(nb, npr), conf_t, lossl)

    return (outl[0, 0], outc[0, 0])
